# Initial kernel scaffold; baseline (speedup 1.0000x reference)
import jax
import jax.numpy as jnp
from jax.experimental import pallas as pl


def kernel(x_nchw, w1, b1, w2, b2, gamma, beta):
    raise NotImplementedError("write your pallas kernel here")


# speedup vs baseline: 1.0132x; 1.0132x over previous
import functools

import jax
import jax.numpy as jnp
from jax import lax
from jax.experimental import pallas as pl
from jax.experimental.pallas import tpu as pltpu

EPS = 1e-5


def _ru(x, m):
    return (x + m - 1) // m * m


def _conv_stats_kernel(xc_ref, w1_ref, b1_ref, w2_ref, b2_ref,
                       h2_ref, stats_ref, h1pad_ref, *, H, W, Mx):
    Wp = W + 2
    M = H * Wp
    Cmid = w1_ref.shape[-1]
    Cout = w2_ref.shape[-1]

    col = lax.broadcasted_iota(jnp.int32, (M, 1), 0) % Wp
    mask = (col < W).astype(jnp.float32)

    acc1 = jnp.dot(xc_ref[0], w1_ref[...], preferred_element_type=jnp.float32)
    h1 = jnp.maximum(acc1 + b1_ref[...], 0.0) * mask

    h1pad_ref[0:Wp + 1, :] = jnp.zeros((Wp + 1, Cmid), jnp.float32)
    h1pad_ref[Wp + 1 + M:Mx, :] = jnp.zeros((Mx - Wp - 1 - M, Cmid),
                                            jnp.float32)
    h1pad_ref[Wp + 1:Wp + 1 + M, :] = h1

    acc2 = jnp.zeros((M, Cout), jnp.float32)
    for dy in range(3):
        for dx in range(3):
            o = dy * Wp + dx
            acc2 = acc2 + jnp.dot(h1pad_ref[o:o + M, :], w2_ref[dy * 3 + dx],
                                  preferred_element_type=jnp.float32)
    h2 = (acc2 + b2_ref[...]) * mask

    h2_ref[0] = h2
    stats_ref[0, 0:1, :] = jnp.sum(h2, axis=0, keepdims=True)
    stats_ref[0, 1:2, :] = jnp.sum(h2 * h2, axis=0, keepdims=True)


def _bn_pool_kernel(h2_ref, sc_ref, sh_ref, feat_ref, pool_ref, *, H, W, C):
    y = jnp.maximum(h2_ref[0] * sc_ref[...] + sh_ref[...], 0.0)

    feat_ref[0] = y[:, :W * C]

    rm = jnp.maximum(y[0:H - 1, :], y[1:H, :])
    r = lax.broadcasted_iota(jnp.int32, (H // 2, H - 1), 0)
    k = lax.broadcasted_iota(jnp.int32, (H // 2, H - 1), 1)
    sel = (k == 2 * r).astype(jnp.float32)
    pa = jnp.dot(sel, rm, preferred_element_type=jnp.float32)

    pb = jnp.maximum(pa[:, 0:W * C], pa[:, C:(W + 1) * C])
    pool = jnp.concatenate(
        [pb[:, (2 * C) * j:(2 * C) * j + C] for j in range(W // 2)], axis=0)
    pool_ref[0] = pool


@jax.jit
def kernel(x_nchw, w1, b1, w2, b2, gamma, beta):
    N, Cin, H, W = x_nchw.shape
    Cmid = w1.shape[-1]
    Cout = w2.shape[-1]
    P, Wp = H + 2, W + 2
    M = H * Wp
    Mx = _ru(P * Wp + 2, 8)

    x = jnp.transpose(x_nchw, (0, 2, 3, 1))
    x_pad = jnp.pad(x, ((0, 0), (1, 1), (1, 1), (0, 0)))
    x_slab = x_pad.reshape(N, P * Wp, Cin)
    x_slab = jnp.pad(x_slab, ((0, 0), (0, Mx - P * Wp), (0, 0)))
    offs = [dy * Wp + dx for dy in range(3) for dx in range(3)]
    x_cols = jnp.concatenate([x_slab[:, o:o + M, :] for o in offs], axis=-1)

    w1c = w1.reshape(9 * Cin, Cmid)
    w2r = w2.reshape(9, Cmid, Cout)

    h2_slab, stats = pl.pallas_call(
        functools.partial(_conv_stats_kernel, H=H, W=W, Mx=Mx),
        out_shape=(
            jax.ShapeDtypeStruct((N, M, Cout), jnp.float32),
            jax.ShapeDtypeStruct((N, 2, Cout), jnp.float32),
        ),
        grid=(N,),
        in_specs=[
            pl.BlockSpec((1, M, 9 * Cin), lambda n: (n, 0, 0)),
            pl.BlockSpec((9 * Cin, Cmid), lambda n: (0, 0)),
            pl.BlockSpec((1, Cmid), lambda n: (0, 0)),
            pl.BlockSpec((9, Cmid, Cout), lambda n: (0, 0, 0)),
            pl.BlockSpec((1, Cout), lambda n: (0, 0)),
        ],
        out_specs=(
            pl.BlockSpec((1, M, Cout), lambda n: (n, 0, 0)),
            pl.BlockSpec((1, 2, Cout), lambda n: (n, 0, 0)),
        ),
        scratch_shapes=[pltpu.VMEM((Mx, Cmid), jnp.float32)],
        compiler_params=pltpu.CompilerParams(
            dimension_semantics=("parallel",)),
    )(x_cols, w1c, b1, w2r, b2)

    count = float(N * H * W)
    tot = jnp.sum(stats, axis=0)
    mean = tot[0] * (1.0 / count)
    var = tot[1] * (1.0 / count) - mean * mean
    inv = lax.rsqrt(var + EPS)
    scale = gamma[0] * inv
    shift = beta[0] - mean * scale
    scW = jnp.tile(scale[None, :], (1, Wp))
    shW = jnp.tile(shift[None, :], (1, Wp))

    h2_v = h2_slab.reshape(N, H, Wp * Cout)
    Ph, Pw = H // 2, W // 2
    feat_hwc, pool_t = pl.pallas_call(
        functools.partial(_bn_pool_kernel, H=H, W=W, C=Cout),
        out_shape=(
            jax.ShapeDtypeStruct((N, H, W * Cout), jnp.float32),
            jax.ShapeDtypeStruct((N, Pw * Ph, Cout), jnp.float32),
        ),
        grid=(N,),
        in_specs=[
            pl.BlockSpec((1, H, Wp * Cout), lambda n: (n, 0, 0)),
            pl.BlockSpec((1, Wp * Cout), lambda n: (0, 0)),
            pl.BlockSpec((1, Wp * Cout), lambda n: (0, 0)),
        ],
        out_specs=(
            pl.BlockSpec((1, H, W * Cout), lambda n: (n, 0, 0)),
            pl.BlockSpec((1, Pw * Ph, Cout), lambda n: (n, 0, 0)),
        ),
        compiler_params=pltpu.CompilerParams(
            dimension_semantics=("parallel",)),
    )(h2_v, scW, shW)

    feat = jnp.transpose(feat_hwc.reshape(N, H, W, Cout), (0, 3, 1, 2))
    pooled = jnp.transpose(pool_t.reshape(N, Pw, Ph, Cout), (0, 3, 2, 1))
    return pooled, feat


# speedup vs baseline: 7.4927x; 7.3951x over previous
import functools

import jax
import jax.numpy as jnp
from jax import lax
from jax.experimental import pallas as pl
from jax.experimental.pallas import tpu as pltpu

EPS = 1e-5


def _ru(x, m):
    return (x + m - 1) // m * m


def _conv_stats_kernel(x_ref, w1_ref, b1_ref, w2_ref, b2_ref,
                       h2_ref, stats_ref, xpad_ref, h1pad_ref, *, H, W, Mx):
    Wp = W + 2
    M = H * Wp
    Cin = x_ref.shape[1]
    Cmid = w1_ref.shape[0]
    Cout = w2_ref.shape[1]

    xpad_ref[0:Cin, :] = jnp.zeros((Cin, Mx), jnp.float32)
    for h in range(H):
        xpad_ref[0:Cin, Wp * (h + 1) + 1:Wp * (h + 1) + 1 + W] = (
            x_ref[0][:, W * h:W * (h + 1)])

    x_cols = jnp.concatenate(
        [xpad_ref[0:Cin, dy * Wp + dx:dy * Wp + dx + M]
         for dy in range(3) for dx in range(3)], axis=0)

    lane = lax.broadcasted_iota(jnp.int32, (1, M), 1) % Wp
    mask = (lane < W).astype(jnp.float32)

    h1 = jnp.maximum(
        jnp.dot(w1_ref[...], x_cols, preferred_element_type=jnp.float32)
        + b1_ref[...], 0.0) * mask

    h1pad_ref[:, 0:Wp + 1] = jnp.zeros((Cmid, Wp + 1), jnp.float32)
    h1pad_ref[:, Wp + 1 + M:Mx] = jnp.zeros((Cmid, Mx - Wp - 1 - M),
                                            jnp.float32)
    h1pad_ref[:, Wp + 1:Wp + 1 + M] = h1

    acc = jnp.zeros((Cout, M), jnp.float32)
    for dy in range(3):
        for dx in range(3):
            o = dy * Wp + dx
            acc = acc + jnp.dot(w2_ref[dy * 3 + dx],
                                h1pad_ref[:, o:o + M],
                                preferred_element_type=jnp.float32)

    h2 = jnp.concatenate(
        [acc[:, Wp * h:Wp * h + W] for h in range(H)], axis=1) + b2_ref[...]

    h2_ref[0] = h2
    stats_ref[0, :, 0:1] = jnp.sum(h2, axis=1, keepdims=True)
    stats_ref[0, :, 1:2] = jnp.sum(h2 * h2, axis=1, keepdims=True)


def _bn_pool_kernel(h2_ref, sc_ref, sh_ref, feat_ref, pool_ref, *, H, W):
    y = jnp.maximum(h2_ref[0] * sc_ref[...] + sh_ref[...], 0.0)
    feat_ref[0] = y

    rm = jnp.maximum(y[:, 0:(H - 1) * W], y[:, W:H * W])
    cm = jnp.maximum(rm[:, 0:(H - 1) * W - 1], rm[:, 1:(H - 1) * W])

    r = lax.broadcasted_iota(jnp.int32, (W - 1, W // 2), 0)
    c = lax.broadcasted_iota(jnp.int32, (W - 1, W // 2), 1)
    sel = (r == 2 * c).astype(jnp.float32)
    pool = jnp.concatenate(
        [jnp.dot(cm[:, 2 * W * i:2 * W * i + W - 1], sel,
                 preferred_element_type=jnp.float32)
         for i in range(H // 2)], axis=1)
    pool_ref[0] = pool


@jax.jit
def kernel(x_nchw, w1, b1, w2, b2, gamma, beta):
    N, Cin, H, W = x_nchw.shape
    Cmid = w1.shape[-1]
    Cout = w2.shape[-1]
    P, Wp = H + 2, W + 2
    M = H * Wp
    Mx = _ru(P * Wp + 2, 8)

    x_flat = x_nchw.reshape(N, Cin, H * W)
    w1t = jnp.transpose(w1.reshape(9 * Cin, Cmid))
    w2t = jnp.transpose(w2.reshape(9, Cmid, Cout), (0, 2, 1))
    b1t = jnp.transpose(b1)
    b2t = jnp.transpose(b2)

    h2c, stats = pl.pallas_call(
        functools.partial(_conv_stats_kernel, H=H, W=W, Mx=Mx),
        out_shape=(
            jax.ShapeDtypeStruct((N, Cout, H * W), jnp.float32),
            jax.ShapeDtypeStruct((N, Cout, 2), jnp.float32),
        ),
        grid=(N,),
        in_specs=[
            pl.BlockSpec((1, Cin, H * W), lambda n: (n, 0, 0)),
            pl.BlockSpec((Cmid, 9 * Cin), lambda n: (0, 0)),
            pl.BlockSpec((Cmid, 1), lambda n: (0, 0)),
            pl.BlockSpec((9, Cout, Cmid), lambda n: (0, 0, 0)),
            pl.BlockSpec((Cout, 1), lambda n: (0, 0)),
        ],
        out_specs=(
            pl.BlockSpec((1, Cout, H * W), lambda n: (n, 0, 0)),
            pl.BlockSpec((1, Cout, 2), lambda n: (n, 0, 0)),
        ),
        scratch_shapes=[
            pltpu.VMEM((8, Mx), jnp.float32),
            pltpu.VMEM((Cmid, Mx), jnp.float32),
        ],
        compiler_params=pltpu.CompilerParams(
            dimension_semantics=("parallel",)),
    )(x_flat, w1t, b1t, w2t, b2t)

    count = float(N * H * W)
    tot = jnp.sum(stats, axis=0)
    mean = tot[:, 0] * (1.0 / count)
    var = tot[:, 1] * (1.0 / count) - mean * mean
    inv = lax.rsqrt(var + EPS)
    scale = gamma[0] * inv
    shift = beta[0] - mean * scale

    Ph, Pw = H // 2, W // 2
    feat_c, pool_c = pl.pallas_call(
        functools.partial(_bn_pool_kernel, H=H, W=W),
        out_shape=(
            jax.ShapeDtypeStruct((N, Cout, H * W), jnp.float32),
            jax.ShapeDtypeStruct((N, Cout, Ph * Pw), jnp.float32),
        ),
        grid=(N,),
        in_specs=[
            pl.BlockSpec((1, Cout, H * W), lambda n: (n, 0, 0)),
            pl.BlockSpec((Cout, 1), lambda n: (0, 0)),
            pl.BlockSpec((Cout, 1), lambda n: (0, 0)),
        ],
        out_specs=(
            pl.BlockSpec((1, Cout, H * W), lambda n: (n, 0, 0)),
            pl.BlockSpec((1, Cout, Ph * Pw), lambda n: (n, 0, 0)),
        ),
        compiler_params=pltpu.CompilerParams(
            dimension_semantics=("parallel",)),
    )(h2c, scale[:, None], shift[:, None])

    feat = feat_c.reshape(N, Cout, H, W)
    pooled = pool_c.reshape(N, Cout, Ph, Pw)
    return pooled, feat


# speedup vs baseline: 7.7562x; 1.0352x over previous
import functools

import jax
import jax.numpy as jnp
from jax import lax
from jax.experimental import pallas as pl
from jax.experimental.pallas import tpu as pltpu

EPS = 1e-5
_B = 4


def _ru(x, m):
    return (x + m - 1) // m * m


def _conv_stats_kernel(x_ref, w1_ref, b1_ref, w2_ref, b2_ref,
                       h2_ref, stats_ref, xpad_ref, h1pad_ref, *, H, W, Mx):
    Wp = W + 2
    M = H * Wp
    Cin = x_ref.shape[1]
    Cmid = w1_ref.shape[0]
    Cout = w2_ref.shape[1]
    B = x_ref.shape[0]

    lane = lax.broadcasted_iota(jnp.int32, (1, M), 1) % Wp
    mask = (lane < W).astype(jnp.float32)

    for b in range(B):
        xpad_ref[b, 0:Cin, :] = jnp.zeros((Cin, Mx), jnp.bfloat16)
        xb = x_ref[b].astype(jnp.bfloat16)
        for h in range(H):
            xpad_ref[b, 0:Cin, Wp * (h + 1) + 1:Wp * (h + 1) + 1 + W] = (
                xb[:, W * h:W * (h + 1)])

        x_cols = jnp.concatenate(
            [xpad_ref[b, 0:Cin, dy * Wp + dx:dy * Wp + dx + M]
             for dy in range(3) for dx in range(3)], axis=0)

        h1 = jnp.maximum(
            jnp.dot(w1_ref[...], x_cols, preferred_element_type=jnp.float32)
            + b1_ref[...], 0.0) * mask

        h1pad_ref[b, :, 0:Wp + 1] = jnp.zeros((Cmid, Wp + 1), jnp.bfloat16)
        h1pad_ref[b, :, Wp + 1 + M:Mx] = jnp.zeros((Cmid, Mx - Wp - 1 - M),
                                                   jnp.bfloat16)
        h1pad_ref[b, :, Wp + 1:Wp + 1 + M] = h1.astype(jnp.bfloat16)

        acc = jnp.zeros((Cout, M), jnp.float32)
        for dy in range(3):
            for dx in range(3):
                o = dy * Wp + dx
                acc = acc + jnp.dot(w2_ref[dy * 3 + dx],
                                    h1pad_ref[b, :, o:o + M],
                                    preferred_element_type=jnp.float32)

        h2 = jnp.concatenate(
            [acc[:, Wp * h:Wp * h + W] for h in range(H)],
            axis=1) + b2_ref[...]

        h2_ref[b] = h2.astype(jnp.bfloat16)
        stats_ref[b, :, 0:1] = jnp.sum(h2, axis=1, keepdims=True)
        stats_ref[b, :, 1:2] = jnp.sum(h2 * h2, axis=1, keepdims=True)


def _bn_pool_kernel(h2_ref, sc_ref, sh_ref, feat_ref, pool_ref, *, H, W):
    B = h2_ref.shape[0]
    r = lax.broadcasted_iota(jnp.int32, (W - 1, W // 2), 0)
    c = lax.broadcasted_iota(jnp.int32, (W - 1, W // 2), 1)
    sel = (r == 2 * c).astype(jnp.float32)

    for b in range(B):
        y = jnp.maximum(h2_ref[b].astype(jnp.float32) * sc_ref[...]
                        + sh_ref[...], 0.0)
        feat_ref[b] = y

        rm = jnp.maximum(y[:, 0:(H - 1) * W], y[:, W:H * W])
        cm = jnp.maximum(rm[:, 0:(H - 1) * W - 1], rm[:, 1:(H - 1) * W])

        pool = jnp.concatenate(
            [jnp.dot(cm[:, 2 * W * i:2 * W * i + W - 1], sel,
                     preferred_element_type=jnp.float32)
             for i in range(H // 2)], axis=1)
        pool_ref[b] = pool


@jax.jit
def kernel(x_nchw, w1, b1, w2, b2, gamma, beta):
    N, Cin, H, W = x_nchw.shape
    Cmid = w1.shape[-1]
    Cout = w2.shape[-1]
    P, Wp = H + 2, W + 2
    M = H * Wp
    Mx = _ru(P * Wp + 2, 8)
    B = _B if N % _B == 0 else 1

    x_flat = x_nchw.reshape(N, Cin, H * W)
    w1t = jnp.transpose(w1.reshape(9 * Cin, Cmid)).astype(jnp.bfloat16)
    w2t = jnp.transpose(w2.reshape(9, Cmid, Cout),
                        (0, 2, 1)).astype(jnp.bfloat16)
    b1t = jnp.transpose(b1)
    b2t = jnp.transpose(b2)

    h2c, stats = pl.pallas_call(
        functools.partial(_conv_stats_kernel, H=H, W=W, Mx=Mx),
        out_shape=(
            jax.ShapeDtypeStruct((N, Cout, H * W), jnp.bfloat16),
            jax.ShapeDtypeStruct((N, Cout, 2), jnp.float32),
        ),
        grid=(N // B,),
        in_specs=[
            pl.BlockSpec((B, Cin, H * W), lambda n: (n, 0, 0)),
            pl.BlockSpec((Cmid, 9 * Cin), lambda n: (0, 0)),
            pl.BlockSpec((Cmid, 1), lambda n: (0, 0)),
            pl.BlockSpec((9, Cout, Cmid), lambda n: (0, 0, 0)),
            pl.BlockSpec((Cout, 1), lambda n: (0, 0)),
        ],
        out_specs=(
            pl.BlockSpec((B, Cout, H * W), lambda n: (n, 0, 0)),
            pl.BlockSpec((B, Cout, 2), lambda n: (n, 0, 0)),
        ),
        scratch_shapes=[
            pltpu.VMEM((B, 8, Mx), jnp.bfloat16),
            pltpu.VMEM((B, Cmid, Mx), jnp.bfloat16),
        ],
        compiler_params=pltpu.CompilerParams(
            dimension_semantics=("parallel",)),
    )(x_flat, w1t, b1t, w2t, b2t)

    count = float(N * H * W)
    tot = jnp.sum(stats, axis=0)
    mean = tot[:, 0] * (1.0 / count)
    var = tot[:, 1] * (1.0 / count) - mean * mean
    inv = lax.rsqrt(var + EPS)
    scale = gamma[0] * inv
    shift = beta[0] - mean * scale

    Ph, Pw = H // 2, W // 2
    feat_c, pool_c = pl.pallas_call(
        functools.partial(_bn_pool_kernel, H=H, W=W),
        out_shape=(
            jax.ShapeDtypeStruct((N, Cout, H * W), jnp.float32),
            jax.ShapeDtypeStruct((N, Cout, Ph * Pw), jnp.float32),
        ),
        grid=(N // B,),
        in_specs=[
            pl.BlockSpec((B, Cout, H * W), lambda n: (n, 0, 0)),
            pl.BlockSpec((Cout, 1), lambda n: (0, 0)),
            pl.BlockSpec((Cout, 1), lambda n: (0, 0)),
        ],
        out_specs=(
            pl.BlockSpec((B, Cout, H * W), lambda n: (n, 0, 0)),
            pl.BlockSpec((B, Cout, Ph * Pw), lambda n: (n, 0, 0)),
        ),
        compiler_params=pltpu.CompilerParams(
            dimension_semantics=("parallel",)),
    )(h2c, scale[:, None], shift[:, None])

    feat = feat_c.reshape(N, Cout, H, W)
    pooled = pool_c.reshape(N, Cout, Ph, Pw)
    return pooled, feat
